# baseline (speedup 1.0000x reference)
import jax
import jax.numpy as jnp
import numpy as np
from jax import lax
from jax.experimental import pallas as pl
from jax.experimental.pallas import tpu as pltpu
from jax.experimental.pallas import tpu_sc as plsc

EMB = 1024
H = 16
HS = 16
K = 4
G = 1
NADD = 4
P = K * (2 + G + NADD)
B = 2
C = 4096
BC = B * C
BH = B * H
NC = 2
NS = 16
CB = 4096
SCB = 256
NBLK = C // SCB


def _sample_offsets():
    u32 = np.uint32

    def rotl(x, r):
        return (x << u32(r)) | (x >> u32(32 - r))

    def tf2x32(k1, k2, x1, x2):
        ks = [u32(k1), u32(k2), u32(k1) ^ u32(k2) ^ u32(0x1BD11BDA)]
        rot = [[13, 15, 26, 6], [17, 29, 16, 24]]
        x = [x1.astype(u32) + ks[0], x2.astype(u32) + ks[1]]
        for i in range(5):
            for r in rot[i % 2]:
                x[0] = x[0] + x[1]
                x[1] = x[0] ^ rotl(x[1], r)
            x[0] = x[0] + ks[(i + 1) % 3]
            x[1] = x[1] + ks[(i + 2) % 3] + u32(i + 1)
        return x[0], x[1]

    def split2(key):
        b1, b2 = tf2x32(key[0], key[1], np.zeros(2, u32),
                        np.arange(2, dtype=u32))
        return (b1[0], b2[0]), (b1[1], b2[1])

    def bits32(key, n):
        b1, b2 = tf2x32(key[0], key[1], np.zeros(n, u32),
                        np.arange(n, dtype=u32))
        return b1 ^ b2

    def randint(key, n, minval, maxval):
        k1, k2 = split2(key)
        hi, lo = bits32(k1, n), bits32(k2, n)
        span = u32(maxval - minval)
        mult = (u32(65536) % span)
        mult = (mult * mult) % span
        off = ((hi % span) * mult + lo % span) % span
        return (np.int64(minval) + off).astype(np.int32)

    r1, r2 = split2((u32(0), u32(42)))
    glob = randint(r1, B * H * C * K * G, 0, C).reshape(B, H, C, K)
    rel = randint(r2, B * H * C * K * NADD, -1, 2).reshape(B, H, C, K * NADD)
    glob_t = np.ascontiguousarray(
        glob.transpose(1, 3, 0, 2).reshape(H, K, BC)).astype(np.float32)
    rel_t = np.ascontiguousarray(
        rel.transpose(1, 3, 0, 2).reshape(H, K * NADD, BC)).astype(np.float32)
    return glob_t, rel_t


_GLOB_T, _REL_T = _sample_offsets()


def _proj_body(w_ref, x_ref, q_ref, k_ref, v_ref, m_ref, s_ref):
    acc = jnp.dot(x_ref[...], w_ref[...], preferred_element_type=jnp.float32)
    acc_t = acc.T
    q_ref[...] = acc_t[0:256, :]
    m_ref[...] = acc_t[768:832, :]
    s_ref[...] = acc_t[832:896, :]

    def pack_pairs(rows):
        ri = lax.bitcast_convert_type(rows, jnp.uint32) + 0x8000
        r3 = ri.reshape(128, 2, ri.shape[1])
        pp = (r3[:, 0, :] & jnp.uint32(0xFFFF0000)) | (r3[:, 1, :] >> 16)
        return lax.bitcast_convert_type(pp, jnp.int32)

    k_ref[...] = pack_pairs(acc_t[256:512, :])
    v_ref[...] = pack_pairs(acc_t[512:768, :])


def _proj(x2, wcat, bm=1024):
    return pl.pallas_call(
        _proj_body,
        grid=(BC // bm,),
        in_specs=[pl.BlockSpec((EMB, 896), lambda i: (0, 0)),
                  pl.BlockSpec((bm, EMB), lambda i: (i, 0))],
        out_specs=[pl.BlockSpec((256, bm), lambda i: (0, i)),
                   pl.BlockSpec((128, bm), lambda i: (0, i)),
                   pl.BlockSpec((128, bm), lambda i: (0, i)),
                   pl.BlockSpec((64, bm), lambda i: (0, i)),
                   pl.BlockSpec((64, bm), lambda i: (0, i))],
        out_shape=[jax.ShapeDtypeStruct((256, BC), jnp.float32),
                   jax.ShapeDtypeStruct((128, BC), jnp.int32),
                   jax.ShapeDtypeStruct((128, BC), jnp.int32),
                   jax.ShapeDtypeStruct((64, BC), jnp.float32),
                   jax.ShapeDtypeStruct((64, BC), jnp.float32)],
    )(wcat, x2)


def _build_points(m, g, r):
    fl = jnp.floor(m)
    relr = r.reshape(K, NADD, CB)
    pts3 = jnp.concatenate(
        [fl[:, None, :], fl[:, None, :] + 1.0, g[:, None, :],
         fl[:, None, :] + relr], axis=1)
    pts = pts3.reshape(P, CB)
    return jnp.clip(pts, 0.0, float(C - 1))


def _dens_dup(idxi, m, sg):
    idxf = idxi.astype(jnp.float32)
    rs = 1.0 / sg
    z = (idxf[:, None, :] - m[None, :, :]) * rs[None, :, :]
    dens = jnp.exp(-0.5 * z * z)
    row = lax.broadcasted_iota(jnp.int32, (P, P, 1), 0)
    col = lax.broadcasted_iota(jnp.int32, (P, P, 1), 1)
    eq = (idxi[:, None, :] == idxi[None, :, :]) & (col < row)
    dup = jnp.any(eq, axis=1)
    return jnp.where(dup[:, None, :], 0.0, dens)


def _stats_body(m_ref, s_ref, g_ref, r_ref, mv_ref, idx_ref, w_ref,
                dens_s, idx_s, sums_s):
    pp = pl.program_id(2)
    j = pl.program_id(3)

    @pl.when(pp == 0)
    def _():
        m = m_ref[0]
        pts = _build_points(m, g_ref[0], r_ref[0])
        idxi = pts.astype(jnp.int32)
        idx_ref[0] = idxi
        idx_s[:, pl.ds(j * CB, CB)] = idxi
        densz = _dens_dup(idxi, m, s_ref[0])
        dens_s[:, :, pl.ds(j * CB, CB)] = densz
        part = jnp.sum(densz, axis=2)
        w_ref[0] = jnp.zeros((P, CB), jnp.float32)

        @pl.when(j == 0)
        def _():
            sums_s[...] = part

        @pl.when(j != 0)
        def _():
            sums_s[...] = sums_s[...] + part

    @pl.when(pp == 1)
    def _():
        idx_ref[0] = idx_s[:, pl.ds(j * CB, CB)]
        densz = dens_s[:, :, pl.ds(j * CB, CB)]
        coef = mv_ref[...] / (sums_s[...] + 1e-8)
        w_ref[0] = jnp.sum(densz * coef[:, :, None], axis=1)


def _stats(mt, st, gt, rt, mv):
    nj = C // CB
    iix = lambda h, b, pp, j: (h, 0, b * nj + j)
    return pl.pallas_call(
        _stats_body,
        grid=(H, B, 2, nj),
        in_specs=[pl.BlockSpec((1, K, CB), iix),
                  pl.BlockSpec((1, K, CB), iix),
                  pl.BlockSpec((1, K, CB), iix),
                  pl.BlockSpec((1, K * NADD, CB), iix),
                  pl.BlockSpec((1, K), lambda h, b, pp, j: (0, 0))],
        out_specs=[pl.BlockSpec((1, P, CB), iix),
                   pl.BlockSpec((1, P, CB), iix)],
        out_shape=[jax.ShapeDtypeStruct((H, P, BC), jnp.int32),
                   jax.ShapeDtypeStruct((H, P, BC), jnp.float32)],
        scratch_shapes=[pltpu.VMEM((P, K, C), jnp.float32),
                        pltpu.VMEM((P, C), jnp.int32),
                        pltpu.VMEM((P, K), jnp.float32)],
    )(mt, st, gt, rt, mv.reshape(1, K))


def _sc_att_body(q_hbm, k_hbm, v_hbm, idx_hbm, w_hbm, out_hbm,
                 ktab, vtab, qblk, iblk, wblk, nwblk, oblk, insem, outsem):
    wid = lax.axis_index("s") * NC + lax.axis_index("c")
    h = wid // B
    b = wid - h * B
    pltpu.sync_copy(k_hbm.at[pl.ds(h * 8, 8), pl.ds(b * C, C)], ktab)
    pltpu.sync_copy(v_hbm.at[pl.ds(h * 8, 8), pl.ds(b * C, C)], vtab)
    kmask = jnp.full((16,), -65536, jnp.int32)
    dv8 = [jnp.full((16,), d, jnp.int32) for d in range(8)]

    def in_pairs(i, s):
        col0 = b * C + i * SCB
        return [(q_hbm.at[pl.ds(h * HS, HS), pl.ds(col0, SCB)], qblk.at[s]),
                (idx_hbm.at[h, :, pl.ds(col0, SCB)], iblk.at[s]),
                (w_hbm.at[h, :, pl.ds(col0, SCB)], wblk.at[s])]

    for src, dst in in_pairs(0, 0):
        pltpu.async_copy(src, dst, insem)

    def blk(i, _):
        s = i % 2
        col0 = b * C + i * SCB
        for src, dst in in_pairs(i, s):
            pltpu.make_async_copy(src, dst, insem).wait()

        @pl.when(i + 1 < NBLK)
        def _():
            for src, dst in in_pairs(i + 1, (i + 1) % 2):
                pltpu.async_copy(src, dst, insem)

        @pl.when(i >= 2)
        def _():
            pltpu.make_async_copy(
                oblk.at[s],
                out_hbm.at[pl.ds(h * HS, HS), pl.ds(col0 - 2 * SCB, SCB)],
                outsem).wait()

        def grp_one(g):
            qv = [qblk[s, d, pl.ds(g * 16, 16)] for d in range(HS)]
            t = []
            for p in range(P):
                ip = iblk[s, p, pl.ds(g * 16, 16)]
                dp = jnp.zeros((16,), jnp.float32)
                for dd in range(8):
                    kw = plsc.load_gather(ktab, [dv8[dd], ip])
                    dp = dp + qv[2 * dd] * plsc.bitcast(kw & kmask,
                                                       jnp.float32)
                    dp = dp + qv[2 * dd + 1] * plsc.bitcast(kw << 16,
                                                            jnp.float32)
                t.append(wblk[s, p, pl.ds(g * 16, 16)] * dp)
            mx = t[0]
            for p in range(1, P):
                mx = jnp.maximum(mx, t[p])
            e = [jnp.exp(tp - mx) for tp in t]
            su = e[0]
            for p in range(1, P):
                su = su + e[p]
            rec = 1.0 / su
            for p in range(P):
                nwblk[p, pl.ds(g * 16, 16)] = e[p] * rec

        def grp(g2, _):
            grp_one(g2 * 2)
            grp_one(g2 * 2 + 1)
            return 0

        lax.fori_loop(0, SCB // 32, grp, 0)

        def grp2_one(g):
            acc = [jnp.zeros((16,), jnp.float32) for _ in range(HS)]
            for p in range(P):
                ip = iblk[s, p, pl.ds(g * 16, 16)]
                nwp = nwblk[p, pl.ds(g * 16, 16)]
                for dd in range(8):
                    vw = plsc.load_gather(vtab, [dv8[dd], ip])
                    acc[2 * dd] = acc[2 * dd] + nwp * plsc.bitcast(
                        vw & kmask, jnp.float32)
                    acc[2 * dd + 1] = acc[2 * dd + 1] + nwp * plsc.bitcast(
                        vw << 16, jnp.float32)
            for d in range(HS):
                oblk[s, d, pl.ds(g * 16, 16)] = acc[d]

        def grp2(g2, _):
            grp2_one(g2 * 2)
            grp2_one(g2 * 2 + 1)
            return 0

        lax.fori_loop(0, SCB // 32, grp2, 0)
        pltpu.async_copy(
            oblk.at[s], out_hbm.at[pl.ds(h * HS, HS), pl.ds(col0, SCB)],
            outsem)
        return 0

    lax.fori_loop(0, NBLK, blk, 0)
    for tail in (NBLK - 2, NBLK - 1):
        pltpu.make_async_copy(
            oblk.at[tail % 2],
            out_hbm.at[pl.ds(h * HS, HS),
                       pl.ds(b * C + tail * SCB, SCB)],
            outsem).wait()


def _sc_attention(qt, kt, vp, idxt, wt):
    mesh = plsc.VectorSubcoreMesh(core_axis_name="c", subcore_axis_name="s",
                                  num_cores=NC, num_subcores=NS)
    fn = pl.kernel(
        _sc_att_body,
        out_type=jax.ShapeDtypeStruct((H * HS, BC), jnp.float32),
        mesh=mesh,
        compiler_params=pltpu.CompilerParams(needs_layout_passes=False,
                                             use_tc_tiling_on_sc=True),
        scratch_types=[
            pltpu.VMEM((8, C), jnp.int32),
            pltpu.VMEM((8, C), jnp.int32),
            pltpu.VMEM((2, HS, SCB), jnp.float32),
            pltpu.VMEM((2, P, SCB), jnp.int32),
            pltpu.VMEM((2, P, SCB), jnp.float32),
            pltpu.VMEM((P, SCB), jnp.float32),
            pltpu.VMEM((2, HS, SCB), jnp.float32),
            pltpu.SemaphoreType.DMA,
            pltpu.SemaphoreType.DMA,
        ],
    )
    return fn(qt, kt, vp, idxt, wt)


def _out_body(u_ref, w_ref, b_ref, o_ref):
    o_ref[...] = lax.dot_general(
        u_ref[...], w_ref[...], (((0,), (0,)), ((), ())),
        preferred_element_type=jnp.float32) + b_ref[...]


def _out_mm(ut, wu, bu, bm=1024):
    return pl.pallas_call(
        _out_body,
        grid=(BC // bm,),
        in_specs=[pl.BlockSpec((H * HS, bm), lambda i: (0, i)),
                  pl.BlockSpec((H * HS, EMB), lambda i: (0, 0)),
                  pl.BlockSpec((1, EMB), lambda i: (0, 0))],
        out_specs=pl.BlockSpec((bm, EMB), lambda i: (i, 0)),
        out_shape=jax.ShapeDtypeStruct((BC, EMB), jnp.float32),
    )(ut, wu, bu.reshape(1, EMB))


def kernel(x, attention_mask, Wq, Wk, Wv, Wu, bu, Wm, Ws, mvalues):
    x2 = x.reshape(BC, EMB)
    wcat = jnp.concatenate([Wq * 0.25, Wk * 0.25, Wv, Wm, Ws], axis=1)
    qt, kt, vp, mraw, sraw = _proj(x2, wcat)

    mt = (jax.nn.sigmoid(mraw) * (C - 1)).reshape(H, K, BC)
    st = (jax.nn.softplus(sraw) + 1e-2).reshape(H, K, BC)

    gt = jnp.asarray(_GLOB_T)
    rt = jnp.asarray(_REL_T)
    idxt, wt = _stats(mt, st, gt, rt, mvalues)

    ut = _sc_attention(qt, kt, vp, idxt, wt)
    return _out_mm(ut, Wu, bu).reshape(B, C, EMB)


# speedup vs baseline: 1.0757x; 1.0757x over previous
import jax
import jax.numpy as jnp
import numpy as np
from jax import lax
from jax.experimental import pallas as pl
from jax.experimental.pallas import tpu as pltpu
from jax.experimental.pallas import tpu_sc as plsc

EMB = 1024
H = 16
HS = 16
K = 4
G = 1
NADD = 4
P = K * (2 + G + NADD)
B = 2
C = 4096
BC = B * C
BH = B * H
NC = 2
NS = 16
CB = 4096
SCB = 256
NBLK = C // SCB


def _sample_offsets():
    u32 = np.uint32

    def rotl(x, r):
        return (x << u32(r)) | (x >> u32(32 - r))

    def tf2x32(k1, k2, x1, x2):
        ks = [u32(k1), u32(k2), u32(k1) ^ u32(k2) ^ u32(0x1BD11BDA)]
        rot = [[13, 15, 26, 6], [17, 29, 16, 24]]
        x = [x1.astype(u32) + ks[0], x2.astype(u32) + ks[1]]
        for i in range(5):
            for r in rot[i % 2]:
                x[0] = x[0] + x[1]
                x[1] = x[0] ^ rotl(x[1], r)
            x[0] = x[0] + ks[(i + 1) % 3]
            x[1] = x[1] + ks[(i + 2) % 3] + u32(i + 1)
        return x[0], x[1]

    def split2(key):
        b1, b2 = tf2x32(key[0], key[1], np.zeros(2, u32),
                        np.arange(2, dtype=u32))
        return (b1[0], b2[0]), (b1[1], b2[1])

    def bits32(key, n):
        b1, b2 = tf2x32(key[0], key[1], np.zeros(n, u32),
                        np.arange(n, dtype=u32))
        return b1 ^ b2

    def randint(key, n, minval, maxval):
        k1, k2 = split2(key)
        hi, lo = bits32(k1, n), bits32(k2, n)
        span = u32(maxval - minval)
        mult = (u32(65536) % span)
        mult = (mult * mult) % span
        off = ((hi % span) * mult + lo % span) % span
        return (np.int64(minval) + off).astype(np.int32)

    r1, r2 = split2((u32(0), u32(42)))
    glob = randint(r1, B * H * C * K * G, 0, C).reshape(B, H, C, K)
    rel = randint(r2, B * H * C * K * NADD, -1, 2).reshape(B, H, C, K * NADD)
    glob_t = np.ascontiguousarray(
        glob.transpose(1, 3, 0, 2).reshape(H, K, BC)).astype(np.float32)
    rel_t = np.ascontiguousarray(
        rel.transpose(1, 3, 0, 2).reshape(H, K * NADD, BC)).astype(np.float32)
    return glob_t, rel_t


_GLOB_T, _REL_T = _sample_offsets()


def _proj_body(w_ref, x_ref, q_ref, k_ref, v_ref, m_ref, s_ref):
    acc = jnp.dot(x_ref[...], w_ref[...], preferred_element_type=jnp.float32)
    acc_t = acc.T
    q_ref[...] = acc_t[0:256, :]
    m_ref[...] = acc_t[768:832, :]
    s_ref[...] = acc_t[832:896, :]

    def pack_pairs(rows):
        ri = lax.bitcast_convert_type(rows, jnp.uint32) + 0x8000
        r3 = ri.reshape(128, 2, ri.shape[1])
        pp = (r3[:, 0, :] & jnp.uint32(0xFFFF0000)) | (r3[:, 1, :] >> 16)
        return lax.bitcast_convert_type(pp, jnp.int32)

    k_ref[...] = pack_pairs(acc_t[256:512, :])
    v_ref[...] = pack_pairs(acc_t[512:768, :])


def _proj(x2, wcat, bm=1024):
    return pl.pallas_call(
        _proj_body,
        grid=(BC // bm,),
        in_specs=[pl.BlockSpec((EMB, 896), lambda i: (0, 0)),
                  pl.BlockSpec((bm, EMB), lambda i: (i, 0))],
        out_specs=[pl.BlockSpec((256, bm), lambda i: (0, i)),
                   pl.BlockSpec((128, bm), lambda i: (0, i)),
                   pl.BlockSpec((128, bm), lambda i: (0, i)),
                   pl.BlockSpec((64, bm), lambda i: (0, i)),
                   pl.BlockSpec((64, bm), lambda i: (0, i))],
        out_shape=[jax.ShapeDtypeStruct((256, BC), jnp.float32),
                   jax.ShapeDtypeStruct((128, BC), jnp.int32),
                   jax.ShapeDtypeStruct((128, BC), jnp.int32),
                   jax.ShapeDtypeStruct((64, BC), jnp.float32),
                   jax.ShapeDtypeStruct((64, BC), jnp.float32)],
    )(wcat, x2)


def _build_points(m, g, r):
    fl = jnp.floor(m)
    relr = r.reshape(K, NADD, CB)
    pts3 = jnp.concatenate(
        [fl[:, None, :], fl[:, None, :] + 1.0, g[:, None, :],
         fl[:, None, :] + relr], axis=1)
    pts = pts3.reshape(P, CB)
    return jnp.clip(pts, 0.0, float(C - 1))


def _dens_dup(idxi, m, sg):
    idxf = idxi.astype(jnp.float32)
    rs = 1.0 / sg
    z = (idxf[:, None, :] - m[None, :, :]) * rs[None, :, :]
    dens = jnp.exp(-0.5 * z * z)
    row = lax.broadcasted_iota(jnp.int32, (P, P, 1), 0)
    col = lax.broadcasted_iota(jnp.int32, (P, P, 1), 1)
    eq = (idxi[:, None, :] == idxi[None, :, :]) & (col < row)
    dup = jnp.any(eq, axis=1)
    return jnp.where(dup[:, None, :], 0.0, dens)


def _stats_body(m_ref, s_ref, g_ref, r_ref, mv_ref, idx_ref, w_ref,
                dens_s, idx_s, sums_s):
    pp = pl.program_id(2)
    j = pl.program_id(3)

    @pl.when(pp == 0)
    def _():
        m = m_ref[0]
        pts = _build_points(m, g_ref[0], r_ref[0])
        idxi = pts.astype(jnp.int32)
        idx_ref[0] = idxi
        idx_s[:, pl.ds(j * CB, CB)] = idxi
        densz = _dens_dup(idxi, m, s_ref[0])
        dens_s[:, :, pl.ds(j * CB, CB)] = densz
        part = jnp.sum(densz, axis=2)
        w_ref[0] = jnp.zeros((P, CB), jnp.float32)

        @pl.when(j == 0)
        def _():
            sums_s[...] = part

        @pl.when(j != 0)
        def _():
            sums_s[...] = sums_s[...] + part

    @pl.when(pp == 1)
    def _():
        idx_ref[0] = idx_s[:, pl.ds(j * CB, CB)]
        densz = dens_s[:, :, pl.ds(j * CB, CB)]
        coef = mv_ref[...] / (sums_s[...] + 1e-8)
        w_ref[0] = jnp.sum(densz * coef[:, :, None], axis=1)


def _stats(mt, st, gt, rt, mv):
    nj = C // CB
    iix = lambda h, b, pp, j: (h, 0, b * nj + j)
    return pl.pallas_call(
        _stats_body,
        grid=(H, B, 2, nj),
        in_specs=[pl.BlockSpec((1, K, CB), iix),
                  pl.BlockSpec((1, K, CB), iix),
                  pl.BlockSpec((1, K, CB), iix),
                  pl.BlockSpec((1, K * NADD, CB), iix),
                  pl.BlockSpec((1, K), lambda h, b, pp, j: (0, 0))],
        out_specs=[pl.BlockSpec((1, P, CB), iix),
                   pl.BlockSpec((1, P, CB), iix)],
        out_shape=[jax.ShapeDtypeStruct((H, P, BC), jnp.int32),
                   jax.ShapeDtypeStruct((H, P, BC), jnp.float32)],
        scratch_shapes=[pltpu.VMEM((P, K, C), jnp.float32),
                        pltpu.VMEM((P, C), jnp.int32),
                        pltpu.VMEM((P, K), jnp.float32)],
    )(mt, st, gt, rt, mv.reshape(1, K))


def _sc_att_body(q_hbm, k_hbm, v_hbm, idx_hbm, w_hbm, out_hbm,
                 ktab, vtab, qblk, iblk, wblk, nwblk, oblk, insem, outsem):
    wid = lax.axis_index("s") * NC + lax.axis_index("c")
    h = wid // B
    b = wid - h * B
    pltpu.sync_copy(k_hbm.at[pl.ds(h * 8, 8), pl.ds(b * C, C)], ktab)
    pltpu.sync_copy(v_hbm.at[pl.ds(h * 8, 8), pl.ds(b * C, C)], vtab)
    kmask = jnp.full((16,), -65536, jnp.int32)
    dv8 = [jnp.full((16,), d, jnp.int32) for d in range(8)]

    def in_pairs(i, s):
        col0 = b * C + i * SCB
        return [(q_hbm.at[pl.ds(h * HS, HS), pl.ds(col0, SCB)], qblk.at[s]),
                (idx_hbm.at[h, :, pl.ds(col0, SCB)], iblk.at[s]),
                (w_hbm.at[h, :, pl.ds(col0, SCB)], wblk.at[s])]

    for src, dst in in_pairs(0, 0):
        pltpu.async_copy(src, dst, insem)

    def blk(i, _):
        s = i % 2
        col0 = b * C + i * SCB
        for src, dst in in_pairs(i, s):
            pltpu.make_async_copy(src, dst, insem).wait()

        @pl.when(i + 1 < NBLK)
        def _():
            for src, dst in in_pairs(i + 1, (i + 1) % 2):
                pltpu.async_copy(src, dst, insem)

        @pl.when(i >= 2)
        def _():
            pltpu.make_async_copy(
                oblk.at[s],
                out_hbm.at[pl.ds(h * HS, HS), pl.ds(col0 - 2 * SCB, SCB)],
                outsem).wait()

        def grp(g, _):
            qv = [qblk[s, d, pl.ds(g * 16, 16)] for d in range(HS)]
            t = []
            for p in range(P):
                ip = iblk[s, p, pl.ds(g * 16, 16)]
                dp = jnp.zeros((16,), jnp.float32)
                for dd in range(8):
                    kw = plsc.load_gather(ktab, [dv8[dd], ip])
                    dp = dp + qv[2 * dd] * plsc.bitcast(kw & kmask,
                                                       jnp.float32)
                    dp = dp + qv[2 * dd + 1] * plsc.bitcast(kw << 16,
                                                            jnp.float32)
                t.append(wblk[s, p, pl.ds(g * 16, 16)] * dp)
            mx = t[0]
            for p in range(1, P):
                mx = jnp.maximum(mx, t[p])
            e = [jnp.exp(tp - mx) for tp in t]
            su = e[0]
            for p in range(1, P):
                su = su + e[p]
            rec = 1.0 / su
            for p in range(P):
                nwblk[p, pl.ds(g * 16, 16)] = e[p] * rec
            return 0

        lax.fori_loop(0, SCB // 16, grp, 0)

        def grp2(g, _):
            acc = [jnp.zeros((16,), jnp.float32) for _ in range(HS)]
            for p in range(P):
                ip = iblk[s, p, pl.ds(g * 16, 16)]
                nwp = nwblk[p, pl.ds(g * 16, 16)]
                for dd in range(8):
                    vw = plsc.load_gather(vtab, [dv8[dd], ip])
                    acc[2 * dd] = acc[2 * dd] + nwp * plsc.bitcast(
                        vw & kmask, jnp.float32)
                    acc[2 * dd + 1] = acc[2 * dd + 1] + nwp * plsc.bitcast(
                        vw << 16, jnp.float32)
            for d in range(HS):
                oblk[s, d, pl.ds(g * 16, 16)] = acc[d]
            return 0

        lax.fori_loop(0, SCB // 16, grp2, 0)
        pltpu.async_copy(
            oblk.at[s], out_hbm.at[pl.ds(h * HS, HS), pl.ds(col0, SCB)],
            outsem)
        return 0

    lax.fori_loop(0, NBLK, blk, 0)
    for tail in (NBLK - 2, NBLK - 1):
        pltpu.make_async_copy(
            oblk.at[tail % 2],
            out_hbm.at[pl.ds(h * HS, HS),
                       pl.ds(b * C + tail * SCB, SCB)],
            outsem).wait()


def _sc_attention(qt, kt, vp, idxt, wt):
    mesh = plsc.VectorSubcoreMesh(core_axis_name="c", subcore_axis_name="s",
                                  num_cores=NC, num_subcores=NS)
    fn = pl.kernel(
        _sc_att_body,
        out_type=jax.ShapeDtypeStruct((H * HS, BC), jnp.float32),
        mesh=mesh,
        compiler_params=pltpu.CompilerParams(needs_layout_passes=False,
                                             use_tc_tiling_on_sc=True),
        scratch_types=[
            pltpu.VMEM((8, C), jnp.int32),
            pltpu.VMEM((8, C), jnp.int32),
            pltpu.VMEM((2, HS, SCB), jnp.float32),
            pltpu.VMEM((2, P, SCB), jnp.int32),
            pltpu.VMEM((2, P, SCB), jnp.float32),
            pltpu.VMEM((P, SCB), jnp.float32),
            pltpu.VMEM((2, HS, SCB), jnp.float32),
            pltpu.SemaphoreType.DMA,
            pltpu.SemaphoreType.DMA,
        ],
    )
    return fn(qt, kt, vp, idxt, wt)


def _out_body(u_ref, w_ref, b_ref, o_ref):
    o_ref[...] = lax.dot_general(
        u_ref[...], w_ref[...], (((0,), (0,)), ((), ())),
        preferred_element_type=jnp.float32) + b_ref[...]


def _out_mm(ut, wu, bu, bm=1024):
    return pl.pallas_call(
        _out_body,
        grid=(BC // bm,),
        in_specs=[pl.BlockSpec((H * HS, bm), lambda i: (0, i)),
                  pl.BlockSpec((H * HS, EMB), lambda i: (0, 0)),
                  pl.BlockSpec((1, EMB), lambda i: (0, 0))],
        out_specs=pl.BlockSpec((bm, EMB), lambda i: (i, 0)),
        out_shape=jax.ShapeDtypeStruct((BC, EMB), jnp.float32),
    )(ut, wu, bu.reshape(1, EMB))


def kernel(x, attention_mask, Wq, Wk, Wv, Wu, bu, Wm, Ws, mvalues):
    x2 = x.reshape(BC, EMB)
    wcat = jnp.concatenate([Wq * 0.25, Wk * 0.25, Wv, Wm, Ws], axis=1)
    qt, kt, vp, mraw, sraw = _proj(x2, wcat)

    mt = (jax.nn.sigmoid(mraw) * (C - 1)).reshape(H, K, BC)
    st = (jax.nn.softplus(sraw) + 1e-2).reshape(H, K, BC)

    gt = jnp.asarray(_GLOB_T)
    rt = jnp.asarray(_REL_T)
    idxt, wt = _stats(mt, st, gt, rt, mvalues)

    ut = _sc_attention(qt, kt, vp, idxt, wt)
    return _out_mm(ut, Wu, bu).reshape(B, C, EMB)
